# baseline (speedup 1.0000x reference)
import jax
import jax.numpy as jnp
from jax import lax
from jax.experimental import pallas as pl
from jax.experimental.pallas import tpu as pltpu


def _blend_copy_kernel(pos_ref, fidx_ref, seq_ref, tmpl_ref, edge_ref,
                       refer_ref, out_ref):
    del fidx_ref
    f = pl.program_id(0)
    attacked = pos_ref[f] >= 0

    @pl.when(attacked)
    def _():
        e = edge_ref[...]
        t = tmpl_ref[...]
        r = refer_ref[...]
        out_ref[...] = jax.nn.sigmoid(t) * e + r * (1.0 - e)

    @pl.when(jnp.logical_not(attacked))
    def _():
        out_ref[...] = seq_ref[...]


def kernel(seq, refer_seq, attack_index, edge_mask, template):
    _, F, H, W = seq.shape
    K = attack_index.shape[0]
    ai = attack_index.astype(jnp.int32)

    pos = jnp.full((F,), -1, jnp.int32).at[ai].set(
        jnp.arange(K, dtype=jnp.int32), mode="drop")
    arange_f = jnp.arange(F, dtype=jnp.int32)
    last_valid = lax.cummax(jnp.where(pos >= 0, arange_f, -1))
    fidx = jnp.maximum(pos[jnp.clip(last_valid, 0, F - 1)], 0)

    grid_spec = pltpu.PrefetchScalarGridSpec(
        num_scalar_prefetch=2,
        grid=(F,),
        in_specs=[
            pl.BlockSpec((1, 1, H, W), lambda f, pos, fidx: (0, f, 0, 0)),
            pl.BlockSpec((1, 1, H, W), lambda f, pos, fidx: (0, fidx[f], 0, 0)),
            pl.BlockSpec((1, 1, H, W), lambda f, pos, fidx: (0, fidx[f], 0, 0)),
            pl.BlockSpec((1, 1, H, W), lambda f, pos, fidx: (0, fidx[f], 0, 0)),
        ],
        out_specs=pl.BlockSpec((1, 1, H, W), lambda f, pos, fidx: (0, f, 0, 0)),
    )
    return pl.pallas_call(
        _blend_copy_kernel,
        grid_spec=grid_spec,
        out_shape=jax.ShapeDtypeStruct(seq.shape, seq.dtype),
    )(pos, fidx, seq, template, edge_mask, refer_seq)
